# baseline (speedup 1.0000x reference)
import functools

import jax
import jax.numpy as jnp
from jax import lax
from jax.experimental import pallas as pl
from jax.experimental.pallas import tpu as pltpu
from jax.experimental.pallas import tpu_sc as plsc

_LN2 = 0.6931471805599453


def _matmul_kernel(x_ref, w_ref, logits_ref):
    logits_ref[...] = jax.lax.dot_general(
        x_ref[...], w_ref[...], (((1,), (0,)), ((), ())),
        preferred_element_type=jnp.float32)


def _ln(x):
    bits = plsc.bitcast(x, jnp.int32)
    e = ((bits >> 23) & 0xFF) - 127
    m = plsc.bitcast((bits & 0x007FFFFF) | 0x3F800000, jnp.float32)
    s = (m - 1.0) / (m + 1.0)
    s2 = s * s
    lnm = 2.0 * s * (1.0 + s2 * (1.0 / 3.0 + s2 * (1.0 / 5.0 + s2 / 7.0)))
    return e.astype(jnp.float32) * _LN2 + lnm


def _route_body(logits_hbm, ael_hbm, gid_hbm, ew_hbm, ls_hbm, es_hbm,
                buf, aelb, gidb, ewb, lacc, esb, *, TW, G, E):
    NE = G * E
    C = NE + G
    wid = lax.axis_index("s") * 2 + lax.axis_index("c")
    base = wid * TW
    pltpu.sync_copy(logits_hbm.at[pl.ds(base * C, TW * C)], buf)
    zero16 = jnp.zeros((16,), jnp.float32)
    for j in range(NE):
        lacc[pl.ds(j * 16, 16)] = zero16

    iota = lax.iota(jnp.int32, 16)

    def chunk(ch, carry):
        entacc, zacc = carry
        rows = ch * 16 + iota
        rbase = rows * C
        g = jnp.zeros((16,), jnp.int32)
        gm = plsc.load_gather(buf, [rbase + NE])
        for k in range(1, G):
            vk = plsc.load_gather(buf, [rbase + (NE + k)])
            take = vk > gm
            g = jnp.where(take, k, g)
            gm = jnp.where(take, vk, gm)
        gcol = rbase + g * E
        vals = [plsc.load_gather(buf, [gcol + e]) for e in range(E)]
        lid = jnp.zeros((16,), jnp.int32)
        vm = vals[0]
        for e in range(1, E):
            take = vals[e] > vm
            lid = jnp.where(take, e, lid)
            vm = jnp.where(take, vals[e], vm)
        gid = g * E + lid
        egs = [jnp.exp(v - vm) for v in vals]
        ssum = egs[0]
        for e in range(1, E):
            ssum = ssum + egs[e]
        ew = 1.0 / ssum
        m2 = jnp.maximum(vm, 0.0)
        t = jnp.exp(jnp.minimum(vm, 0.0))
        z = jnp.exp(-m2)
        denom = ssum * t + (NE - E) * z
        wsum = egs[0] * (vals[0] - m2)
        for e in range(1, E):
            wsum = wsum + egs[e] * (vals[e] - m2)
        wsum = t * wsum + (NE - E) * z * (-m2)
        ent = _ln(denom) - wsum / denom
        inv = 1.0 / denom
        tid = t * inv
        zid = z * inv
        lrow = (g * E) * 16 + iota
        for e in range(E):
            plsc.addupdate_scatter(lacc, [lrow + e * 16], egs[e] * tid - zid)
        plsc.store_scatter(gidb, [rows], gid)
        plsc.store_scatter(ewb, [rows], ew)
        abase = rows * NE
        for j in range(NE):
            v = jnp.where(g == (j // E), vals[j % E], 0.0)
            plsc.store_scatter(aelb, [abase + j], v)
        return entacc + ent, zacc + zid

    entacc, zacc = lax.fori_loop(0, TW // 16, chunk, (zero16, zero16))
    esb[pl.ds(0, 16)] = entacc
    esb[pl.ds(16, 16)] = zacc
    pltpu.sync_copy(aelb, ael_hbm.at[pl.ds(base * NE, TW * NE)])
    pltpu.sync_copy(gidb, gid_hbm.at[pl.ds(base, TW)])
    pltpu.sync_copy(ewb, ew_hbm.at[pl.ds(base, TW)])
    pltpu.sync_copy(lacc, ls_hbm.at[wid])
    pltpu.sync_copy(esb, es_hbm.at[wid])


def _finalize_kernel(ls_ref, es_ref, var_ref, ent_ref, *, B, NE):
    colsum = jnp.sum(ls_ref[...], axis=(0, 2))
    zsum = jnp.sum(es_ref[:, 1, :])
    load = (colsum + zsum) / B
    mu = jnp.mean(load)
    var_ref[...] = (jnp.sum((load - mu) ** 2) / (NE - 1)).reshape(1, 1)
    ent_ref[...] = (jnp.sum(es_ref[:, 0, :]) / B).reshape(1, 1)


def kernel(hidden_states, Wg, We):
    B, H = hidden_states.shape
    G, E, _ = We.shape
    NE = G * E
    C = NE + G
    W = jnp.concatenate([We.reshape(NE, H), Wg], axis=0).T

    TB = 1024
    logits = pl.pallas_call(
        _matmul_kernel,
        grid=(B // TB,),
        in_specs=[
            pl.BlockSpec((TB, H), lambda i: (i, 0)),
            pl.BlockSpec((H, C), lambda i: (0, 0)),
        ],
        out_specs=pl.BlockSpec((TB, C), lambda i: (i, 0)),
        out_shape=jax.ShapeDtypeStruct((B, C), jnp.float32),
    )(hidden_states, W)

    NW = 32
    TW = B // NW
    mesh = plsc.VectorSubcoreMesh(core_axis_name="c", subcore_axis_name="s")
    route = functools.partial(
        pl.kernel,
        out_type=[
            jax.ShapeDtypeStruct((B * NE,), jnp.float32),
            jax.ShapeDtypeStruct((B,), jnp.int32),
            jax.ShapeDtypeStruct((B,), jnp.float32),
            jax.ShapeDtypeStruct((NW, NE * 16), jnp.float32),
            jax.ShapeDtypeStruct((NW, 32), jnp.float32),
        ],
        mesh=mesh,
        compiler_params=pltpu.CompilerParams(needs_layout_passes=False),
        scratch_types=[
            pltpu.VMEM((TW * C,), jnp.float32),
            pltpu.VMEM((TW * NE,), jnp.float32),
            pltpu.VMEM((TW,), jnp.int32),
            pltpu.VMEM((TW,), jnp.float32),
            pltpu.VMEM((NE * 16,), jnp.float32),
            pltpu.VMEM((32,), jnp.float32),
        ],
    )(functools.partial(_route_body, TW=TW, G=G, E=E))
    ael, gid, ew, ls, es = route(logits.reshape(B * C))

    var, ent = pl.pallas_call(
        functools.partial(_finalize_kernel, B=B, NE=NE),
        in_specs=[
            pl.BlockSpec((NW, NE, 16), lambda: (0, 0, 0)),
            pl.BlockSpec((NW, 2, 16), lambda: (0, 0, 0)),
        ],
        out_specs=(
            pl.BlockSpec((1, 1), lambda: (0, 0)),
            pl.BlockSpec((1, 1), lambda: (0, 0)),
        ),
        out_shape=(
            jax.ShapeDtypeStruct((1, 1), jnp.float32),
            jax.ShapeDtypeStruct((1, 1), jnp.float32),
        ),
    )(ls.reshape(NW, NE, 16), es.reshape(NW, 2, 16))
    return (ael.reshape(B, NE), gid[:, None], ew[:, None],
            var.reshape(()), ent.reshape(()))


# speedup vs baseline: 1.5739x; 1.5739x over previous
import functools

import jax
import jax.numpy as jnp
from jax.experimental import pallas as pl
from jax.experimental.pallas import tpu as pltpu


def _router_kernel(x_ref, w_ref, ael_ref, gid_ref, ew_ref, var_ref, ent_ref,
                   load_acc, ent_acc, *, nsteps, B, G, E):
    i = pl.program_id(0)
    NE = G * E
    x = x_ref[...]
    w = w_ref[...]
    logits = jax.lax.dot_general(
        x, w, (((1,), (0,)), ((), ())),
        preferred_element_type=jnp.float32)
    le = logits[:, :NE]
    lg = logits[:, NE:NE + G]
    g = jnp.argmax(lg, axis=-1).astype(jnp.int32)
    col = jax.lax.broadcasted_iota(jnp.int32, le.shape, 1)
    mask = (col // E) == g[:, None]
    lm = jnp.where(mask, le, jnp.float32(-1e30))
    gid = jnp.argmax(lm, axis=-1).astype(jnp.int32)
    lmax = jnp.max(lm, axis=-1)[:, None]
    eg = jnp.where(mask, jnp.exp(le - lmax), 0.0)
    s = jnp.sum(eg, axis=-1, keepdims=True)
    ew = 1.0 / s
    ael = jnp.where(mask, le, 0.0)
    m = jnp.maximum(lmax, 0.0)
    t = jnp.exp(lmax - m)
    z = jnp.exp(-m)
    denom = s * t + (NE - E) * z
    wsum = jnp.sum(eg * (le - m), axis=-1, keepdims=True) * t \
        + (NE - E) * z * (-m)
    ent = jnp.log(denom) - wsum / denom

    ael_ref[...] = ael
    gid_ref[...] = gid[:, None]
    ew_ref[...] = ew

    @pl.when(i == 0)
    def _init():
        load_acc[...] = jnp.zeros_like(load_acc)
        ent_acc[...] = jnp.zeros_like(ent_acc)

    probs = jnp.where(mask, eg * t, z) / denom
    load_acc[...] += jnp.sum(probs, axis=0, keepdims=True)
    ent_acc[...] += jnp.sum(ent).reshape(1, 1)

    @pl.when(i == nsteps - 1)
    def _finalize():
        load = load_acc[...] / B
        mu = jnp.mean(load)
        var_ref[...] = (jnp.sum((load - mu) ** 2) / (NE - 1)).reshape(1, 1)
        ent_ref[...] = ent_acc[...] / B


def kernel(hidden_states, Wg, We):
    B, H = hidden_states.shape
    G, E, _ = We.shape
    NE = G * E
    W = jnp.concatenate([We.reshape(NE, H), Wg], axis=0).T
    TB = 1024
    nsteps = B // TB
    out_shape = (
        jax.ShapeDtypeStruct((B, NE), jnp.float32),
        jax.ShapeDtypeStruct((B, 1), jnp.int32),
        jax.ShapeDtypeStruct((B, 1), jnp.float32),
        jax.ShapeDtypeStruct((1, 1), jnp.float32),
        jax.ShapeDtypeStruct((1, 1), jnp.float32),
    )
    ael, gid, ew, var, ent = pl.pallas_call(
        functools.partial(_router_kernel, nsteps=nsteps, B=B, G=G, E=E),
        grid=(nsteps,),
        in_specs=[
            pl.BlockSpec((TB, H), lambda i: (i, 0)),
            pl.BlockSpec((H, NE + G), lambda i: (0, 0)),
        ],
        out_specs=(
            pl.BlockSpec((TB, NE), lambda i: (i, 0)),
            pl.BlockSpec((TB, 1), lambda i: (i, 0)),
            pl.BlockSpec((TB, 1), lambda i: (i, 0)),
            pl.BlockSpec((1, 1), lambda i: (0, 0)),
            pl.BlockSpec((1, 1), lambda i: (0, 0)),
        ),
        out_shape=out_shape,
        scratch_shapes=[pltpu.VMEM((1, NE), jnp.float32),
                        pltpu.VMEM((1, 1), jnp.float32)],
    )(hidden_states, W)
    return (ael, gid, ew, var.reshape(()), ent.reshape(()))


# speedup vs baseline: 1.6425x; 1.0436x over previous
import functools

import jax
import jax.numpy as jnp
from jax.experimental import pallas as pl
from jax.experimental.pallas import tpu as pltpu


def _router_kernel(x_ref, w_ref, ael_ref, gid_ref, ew_ref, var_ref, ent_ref,
                   load_acc, ent_acc, *, nsteps, B, G, E):
    i = pl.program_id(0)
    NE = G * E
    x = x_ref[...]
    w = w_ref[...]
    logits = jax.lax.dot_general(
        x, w, (((1,), (1,)), ((), ())),
        preferred_element_type=jnp.float32)
    le = logits[:, :NE]
    lg = logits[:, NE:NE + G]
    g = jnp.argmax(lg, axis=-1).astype(jnp.int32)
    col = jax.lax.broadcasted_iota(jnp.int32, le.shape, 1)
    mask = (col // E) == g[:, None]
    lm = jnp.where(mask, le, jnp.float32(-1e30))
    gid = jnp.argmax(lm, axis=-1).astype(jnp.int32)
    lmax = jnp.max(lm, axis=-1)[:, None]
    eg = jnp.where(mask, jnp.exp(le - lmax), 0.0)
    s = jnp.sum(eg, axis=-1, keepdims=True)
    ew = 1.0 / s
    ael = jnp.where(mask, le, 0.0)
    m = jnp.maximum(lmax, 0.0)
    t = jnp.exp(lmax - m)
    z = jnp.exp(-m)
    denom = s * t + (NE - E) * z
    wsum = jnp.sum(eg * (le - m), axis=-1, keepdims=True) * t \
        + (NE - E) * z * (-m)
    ent = jnp.log(denom) - wsum / denom

    ael_ref[...] = ael
    gid_ref[...] = gid[:, None]
    ew_ref[...] = ew

    @pl.when(i == 0)
    def _init():
        load_acc[...] = jnp.zeros_like(load_acc)
        ent_acc[...] = jnp.zeros_like(ent_acc)

    probs = jnp.where(mask, eg * t, z) / denom
    load_acc[...] += jnp.sum(probs, axis=0, keepdims=True)
    ent_acc[...] += jnp.sum(ent).reshape(1, 1)

    @pl.when(i == nsteps - 1)
    def _finalize():
        load = load_acc[...] / B
        mu = jnp.mean(load)
        var_ref[...] = (jnp.sum((load - mu) ** 2) / (NE - 1)).reshape(1, 1)
        ent_ref[...] = ent_acc[...] / B


def kernel(hidden_states, Wg, We):
    B, H = hidden_states.shape
    G, E, _ = We.shape
    NE = G * E
    W = jnp.concatenate([We.reshape(NE, H), Wg], axis=0)
    TB = 1024
    nsteps = B // TB
    out_shape = (
        jax.ShapeDtypeStruct((B, NE), jnp.float32),
        jax.ShapeDtypeStruct((B, 1), jnp.int32),
        jax.ShapeDtypeStruct((B, 1), jnp.float32),
        jax.ShapeDtypeStruct((1, 1), jnp.float32),
        jax.ShapeDtypeStruct((1, 1), jnp.float32),
    )
    ael, gid, ew, var, ent = pl.pallas_call(
        functools.partial(_router_kernel, nsteps=nsteps, B=B, G=G, E=E),
        grid=(nsteps,),
        in_specs=[
            pl.BlockSpec((TB, H), lambda i: (i, 0)),
            pl.BlockSpec((NE + G, H), lambda i: (0, 0)),
        ],
        out_specs=(
            pl.BlockSpec((TB, NE), lambda i: (i, 0)),
            pl.BlockSpec((TB, 1), lambda i: (i, 0)),
            pl.BlockSpec((TB, 1), lambda i: (i, 0)),
            pl.BlockSpec((1, 1), lambda i: (0, 0)),
            pl.BlockSpec((1, 1), lambda i: (0, 0)),
        ),
        out_shape=out_shape,
        scratch_shapes=[pltpu.VMEM((1, NE), jnp.float32),
                        pltpu.VMEM((1, 1), jnp.float32)],
    )(hidden_states, W)
    return (ael, gid, ew, var.reshape(()), ent.reshape(()))


# speedup vs baseline: 1.7497x; 1.0653x over previous
import functools

import jax
import jax.numpy as jnp
from jax.experimental import pallas as pl
from jax.experimental.pallas import tpu as pltpu


def _router_kernel(x_ref, we_ref, wg_ref, ael_ref, gid_ref, ew_ref, var_ref,
                   ent_ref, load_acc, ent_acc, *, nsteps, B, G, E):
    i = pl.program_id(0)
    NE = G * E
    x = x_ref[...]
    dims = (((1,), (1,)), ((), ()))
    le = jax.lax.dot_general(x, we_ref[...], dims,
                             preferred_element_type=jnp.float32)
    lg = jax.lax.dot_general(x, wg_ref[...], dims,
                             preferred_element_type=jnp.float32)
    g = jnp.argmax(lg, axis=-1).astype(jnp.int32)
    col = jax.lax.broadcasted_iota(jnp.int32, le.shape, 1)
    mask = (col // E) == g[:, None]
    lm = jnp.where(mask, le, jnp.float32(-1e30))
    gid = jnp.argmax(lm, axis=-1).astype(jnp.int32)
    lmax = jnp.max(lm, axis=-1)[:, None]
    eg = jnp.where(mask, jnp.exp(le - lmax), 0.0)
    s = jnp.sum(eg, axis=-1, keepdims=True)
    ew = 1.0 / s
    ael = jnp.where(mask, le, 0.0)
    m = jnp.maximum(lmax, 0.0)
    t = jnp.exp(lmax - m)
    z = jnp.exp(-m)
    denom = s * t + (NE - E) * z
    wsum = jnp.sum(eg * (le - m), axis=-1, keepdims=True) * t \
        + (NE - E) * z * (-m)
    ent = jnp.log(denom) - wsum / denom

    ael_ref[...] = ael
    gid_ref[...] = gid[:, None]
    ew_ref[...] = ew

    @pl.when(i == 0)
    def _init():
        load_acc[...] = jnp.zeros_like(load_acc)
        ent_acc[...] = jnp.zeros_like(ent_acc)

    probs = jnp.where(mask, eg * t, z) / denom
    load_acc[...] += jnp.sum(probs, axis=0, keepdims=True)
    ent_acc[...] += jnp.sum(ent).reshape(1, 1)

    @pl.when(i == nsteps - 1)
    def _finalize():
        load = load_acc[...] / B
        mu = jnp.mean(load)
        var_ref[...] = (jnp.sum((load - mu) ** 2) / (NE - 1)).reshape(1, 1)
        ent_ref[...] = ent_acc[...] / B


def kernel(hidden_states, Wg, We):
    B, H = hidden_states.shape
    G, E, _ = We.shape
    NE = G * E
    TB = 1024
    nsteps = B // TB
    out_shape = (
        jax.ShapeDtypeStruct((B, NE), jnp.float32),
        jax.ShapeDtypeStruct((B, 1), jnp.int32),
        jax.ShapeDtypeStruct((B, 1), jnp.float32),
        jax.ShapeDtypeStruct((1, 1), jnp.float32),
        jax.ShapeDtypeStruct((1, 1), jnp.float32),
    )
    ael, gid, ew, var, ent = pl.pallas_call(
        functools.partial(_router_kernel, nsteps=nsteps, B=B, G=G, E=E),
        grid=(nsteps,),
        in_specs=[
            pl.BlockSpec((TB, H), lambda i: (i, 0)),
            pl.BlockSpec((NE, H), lambda i: (0, 0)),
            pl.BlockSpec((G, H), lambda i: (0, 0)),
        ],
        out_specs=(
            pl.BlockSpec((TB, NE), lambda i: (i, 0)),
            pl.BlockSpec((TB, 1), lambda i: (i, 0)),
            pl.BlockSpec((TB, 1), lambda i: (i, 0)),
            pl.BlockSpec((1, 1), lambda i: (0, 0)),
            pl.BlockSpec((1, 1), lambda i: (0, 0)),
        ),
        out_shape=out_shape,
        scratch_shapes=[pltpu.VMEM((1, NE), jnp.float32),
                        pltpu.VMEM((1, 1), jnp.float32)],
    )(hidden_states, We.reshape(NE, H), Wg)
    return (ael, gid, ew, var.reshape(()), ent.reshape(()))


# speedup vs baseline: 1.7730x; 1.0133x over previous
import functools

import jax
import jax.numpy as jnp
from jax.experimental import pallas as pl
from jax.experimental.pallas import tpu as pltpu


def _router_kernel(x_ref, we_ref, wg_ref, ael_ref, gid_ref, ew_ref, var_ref,
                   ent_ref, load_acc, ent_acc, *, nsteps, B, G, E):
    i = pl.program_id(0)
    NE = G * E
    x = x_ref[...]
    dims = (((1,), (1,)), ((), ()))
    le = jax.lax.dot_general(x, we_ref[...], dims,
                             preferred_element_type=jnp.float32)
    lg = jax.lax.dot_general(x, wg_ref[...], dims,
                             preferred_element_type=jnp.float32)
    g = jnp.argmax(lg, axis=-1).astype(jnp.int32)
    col = jax.lax.broadcasted_iota(jnp.int32, le.shape, 1)
    mask = (col // E) == g[:, None]
    lm = jnp.where(mask, le, jnp.float32(-1e30))
    gid = jnp.argmax(lm, axis=-1).astype(jnp.int32)
    lmax = jnp.max(lm, axis=-1)[:, None]
    eg = jnp.where(mask, jnp.exp(le - lmax), 0.0)
    s = jnp.sum(eg, axis=-1, keepdims=True)
    ew = 1.0 / s
    ael = jnp.where(mask, le, 0.0)
    m = jnp.maximum(lmax, 0.0)
    t = jnp.exp(lmax - m)
    z = jnp.exp(-m)
    denom = s * t + (NE - E) * z
    wsum = jnp.sum(eg * (le - m), axis=-1, keepdims=True) * t \
        + (NE - E) * z * (-m)
    ent = jnp.log(denom) - wsum / denom

    ael_ref[...] = ael
    gid_ref[...] = gid[:, None]
    ew_ref[...] = ew

    @pl.when(i == 0)
    def _init():
        load_acc[...] = jnp.zeros_like(load_acc)
        ent_acc[...] = jnp.zeros_like(ent_acc)

    probs = jnp.where(mask, eg * t, z) / denom
    load_acc[...] += jnp.sum(probs, axis=0, keepdims=True)
    ent_acc[...] += jnp.sum(ent).reshape(1, 1)

    @pl.when(i == nsteps - 1)
    def _finalize():
        load = load_acc[...] / B
        mu = jnp.mean(load)
        var_ref[...] = (jnp.sum((load - mu) ** 2) / (NE - 1)).reshape(1, 1)
        ent_ref[...] = ent_acc[...] / B


def kernel(hidden_states, Wg, We):
    B, H = hidden_states.shape
    G, E, _ = We.shape
    NE = G * E
    TB = 2048
    nsteps = B // TB
    out_shape = (
        jax.ShapeDtypeStruct((B, NE), jnp.float32),
        jax.ShapeDtypeStruct((B, 1), jnp.int32),
        jax.ShapeDtypeStruct((B, 1), jnp.float32),
        jax.ShapeDtypeStruct((1, 1), jnp.float32),
        jax.ShapeDtypeStruct((1, 1), jnp.float32),
    )
    ael, gid, ew, var, ent = pl.pallas_call(
        functools.partial(_router_kernel, nsteps=nsteps, B=B, G=G, E=E),
        grid=(nsteps,),
        in_specs=[
            pl.BlockSpec((TB, H), lambda i: (i, 0)),
            pl.BlockSpec((NE, H), lambda i: (0, 0)),
            pl.BlockSpec((G, H), lambda i: (0, 0)),
        ],
        out_specs=(
            pl.BlockSpec((TB, NE), lambda i: (i, 0)),
            pl.BlockSpec((TB, 1), lambda i: (i, 0)),
            pl.BlockSpec((TB, 1), lambda i: (i, 0)),
            pl.BlockSpec((1, 1), lambda i: (0, 0)),
            pl.BlockSpec((1, 1), lambda i: (0, 0)),
        ),
        out_shape=out_shape,
        scratch_shapes=[pltpu.VMEM((1, NE), jnp.float32),
                        pltpu.VMEM((1, 1), jnp.float32)],
    )(hidden_states, We.reshape(NE, H), Wg)
    return (ael, gid, ew, var.reshape(()), ent.reshape(()))
